# baseline (speedup 1.0000x reference)
import jax
import jax.numpy as jnp
from jax import lax
from jax.experimental import pallas as pl
from jax.experimental.pallas import tpu as pltpu
from jax.experimental.pallas import tpu_sc as plsc

N = 10000
E = 320000
D_IN = 128
D_H = 64

NC = 2
NS = 16
NW = NC * NS
CHUNK = 128
CH = 80
GRP = 8
E_PAD = NW * CH * CHUNK
N_PAD = 10240
RPT = N_PAD // NS

_mesh = plsc.VectorSubcoreMesh(core_axis_name="c", subcore_axis_name="s")
_sc_params = pltpu.CompilerParams(use_tc_tiling_on_sc=False)


def _deg_body(src_hbm, dst_hbm, ones_hbm, z1_hbm, out_hbm,
              src_v, dst_v, ones_v, dego_sh, degi_sh):
    c = lax.axis_index("c")
    s = lax.axis_index("s")
    wid = c * NS + s
    pltpu.sync_copy(src_hbm.at[wid], src_v)
    pltpu.sync_copy(dst_hbm.at[wid], dst_v)
    pltpu.sync_copy(ones_hbm, ones_v)
    sl = pl.ds(s * RPT, RPT)
    pltpu.sync_copy(z1_hbm.at[sl], dego_sh.at[sl])
    pltpu.sync_copy(z1_hbm.at[sl], degi_sh.at[sl])
    plsc.subcore_barrier()

    @pl.loop(0, CH // GRP)
    def _(j):
        pltpu.sync_copy(ones_v, dego_sh.at[src_v.at[j]], add=True)
        pltpu.sync_copy(ones_v, degi_sh.at[dst_v.at[j]], add=True)

    plsc.subcore_barrier()
    pltpu.sync_copy(dego_sh.at[sl], out_hbm.at[c, 0, sl])
    pltpu.sync_copy(degi_sh.at[sl], out_hbm.at[c, 1, sl])


_deg_call = pl.kernel(
    _deg_body,
    out_type=jax.ShapeDtypeStruct((NC, 2, N_PAD), jnp.float32),
    mesh=_mesh,
    scratch_types=[
        pltpu.VMEM((CH // GRP, GRP * CHUNK), jnp.int32),
        pltpu.VMEM((CH // GRP, GRP * CHUNK), jnp.int32),
        pltpu.VMEM((GRP * CHUNK,), jnp.float32),
        pltpu.VMEM_SHARED((N_PAD,), jnp.float32),
        pltpu.VMEM_SHARED((N_PAD,), jnp.float32),
    ],
    compiler_params=_sc_params,
)


def _layer_body(h_hbm, src_hbm, dst_hbm, zr_hbm, out_hbm,
                src_v, dst_v, rows0_v, agg_sh):
    c = lax.axis_index("c")
    s = lax.axis_index("s")
    wid = c * NS + s
    pltpu.sync_copy(src_hbm.at[wid], src_v)
    pltpu.sync_copy(dst_hbm.at[wid], dst_v)
    sl = pl.ds(s * RPT, RPT)
    pltpu.sync_copy(zr_hbm.at[sl], agg_sh.at[sl])
    plsc.subcore_barrier()

    @pl.loop(0, CH // GRP)
    def _(i):
        pltpu.sync_copy(h_hbm.at[src_v.at[i]], rows0_v)
        pltpu.sync_copy(rows0_v, agg_sh.at[dst_v.at[i]], add=True)

    plsc.subcore_barrier()
    pltpu.sync_copy(agg_sh.at[sl], out_hbm.at[c, sl])


_layer_call = pl.kernel(
    _layer_body,
    out_type=jax.ShapeDtypeStruct((NC, N_PAD, D_H), jnp.float32),
    mesh=_mesh,
    scratch_types=[
        pltpu.VMEM((CH // GRP, GRP * CHUNK), jnp.int32),
        pltpu.VMEM((CH // GRP, GRP * CHUNK), jnp.int32),
        pltpu.VMEM((GRP * CHUNK, D_H), jnp.float32),
        pltpu.VMEM_SHARED((N_PAD, D_H), jnp.float32),
    ],
    compiler_params=_sc_params,
)


def _prep_body(f_ref, w_ref, degp_ref, xw_ref, ns_ref, nd_ref):
    xw_ref[...] = jnp.dot(f_ref[...], w_ref[...],
                          preferred_element_type=jnp.float32,
                          precision=lax.Precision.HIGHEST)
    dego = degp_ref[0, 0:1, :] + degp_ref[1, 0:1, :]
    degi = degp_ref[0, 1:2, :] + degp_ref[1, 1:2, :]
    ns_ref[...] = jnp.where(dego > 0.0, lax.rsqrt(jnp.maximum(dego, 1.0)), 0.0)
    nd_ref[...] = jnp.where(degi > 0.0, lax.rsqrt(jnp.maximum(degi, 1.0)), 0.0)


_prep_call = pl.pallas_call(
    _prep_body,
    out_shape=(
        jax.ShapeDtypeStruct((N_PAD, D_H), jnp.float32),
        jax.ShapeDtypeStruct((1, N_PAD), jnp.float32),
        jax.ShapeDtypeStruct((1, N_PAD), jnp.float32),
    ),
)


def _scale_body(xw_ref, ns_ref, h_ref):
    h_ref[...] = xw_ref[...] * ns_ref[...]


_scale_call = pl.pallas_call(
    _scale_body,
    out_shape=jax.ShapeDtypeStruct((N_PAD, D_H), jnp.float32),
)


def _combine_body(aggp_ref, nd_ref, ns_ref, b_ref, w_ref, h_ref):
    p = aggp_ref[0] + aggp_ref[1]
    x = jnp.maximum(p * nd_ref[...] + b_ref[...], 0.0)
    h_ref[...] = jnp.dot(x, w_ref[...],
                         preferred_element_type=jnp.float32,
                         precision=lax.Precision.HIGHEST) * ns_ref[...]


_combine_call = pl.pallas_call(
    _combine_body,
    out_shape=jax.ShapeDtypeStruct((N_PAD, D_H), jnp.float32),
)


def _final_body(aggp_ref, nd_ref, b_ref, out_ref):
    p = aggp_ref[0] + aggp_ref[1]
    x = jnp.maximum(p * nd_ref[...] + b_ref[...], 0.0)
    out_ref[...] = jnp.sum(x[:N, :], axis=0, keepdims=True) * (1.0 / N)


_final_call = pl.pallas_call(
    _final_body,
    out_shape=jax.ShapeDtypeStruct((1, D_H), jnp.float32),
)


def kernel(features, edge_index, W1, b1, W2, b2, W3, b3):
    src = edge_index[0]
    dst = edge_index[1]
    padv = jnp.full((E_PAD - E,), N, dtype=jnp.int32)
    srcp = jnp.concatenate([src, padv]).reshape(NW, CH // GRP, GRP * CHUNK)
    dstp = jnp.concatenate([dst, padv]).reshape(NW, CH // GRP, GRP * CHUNK)
    featp = jnp.pad(features, ((0, N_PAD - N), (0, 0)))
    ones = jnp.ones((GRP * CHUNK,), jnp.float32)
    z1 = jnp.zeros((N_PAD,), jnp.float32)
    zr = jnp.zeros((N_PAD, D_H), jnp.float32)

    degp = _deg_call(srcp, dstp, ones, z1)
    xw1, ns_row, nd_row = _prep_call(featp, W1, degp)
    ns_col = ns_row.reshape(N_PAD, 1)
    nd_col = nd_row.reshape(N_PAD, 1)

    h = _scale_call(xw1, ns_col)
    for bb, Wn in ((b1, W2), (b2, W3)):
        aggp = _layer_call(h, srcp, dstp, zr)
        h = _combine_call(aggp, nd_col, ns_col, bb.reshape(1, D_H), Wn)
    aggp = _layer_call(h, srcp, dstp, zr)
    out = _final_call(aggp, nd_col, b3.reshape(1, D_H))
    return out.reshape(D_H)


# speedup vs baseline: 2.7867x; 2.7867x over previous
import jax
import jax.numpy as jnp
from jax import lax
from jax.experimental import pallas as pl
from jax.experimental.pallas import tpu as pltpu
from jax.experimental.pallas import tpu_sc as plsc

N = 10000
E = 320000
D_IN = 128
D_H = 64

NC = 2
NS = 16
NW = NC * NS
CHUNK = 128
CH = 80
GRP = 8
E_PAD = NW * CH * CHUNK
N_PAD = 10240
RPT = N_PAD // NS

_mesh = plsc.VectorSubcoreMesh(core_axis_name="c", subcore_axis_name="s")
_sc_params = pltpu.CompilerParams(use_tc_tiling_on_sc=False)


def _deg_body(src_hbm, dst_hbm, ones_hbm, z1_hbm, out_hbm,
              src_v, dst_v, ones_v, dego_sh, degi_sh):
    c = lax.axis_index("c")
    s = lax.axis_index("s")
    wid = c * NS + s
    pltpu.sync_copy(src_hbm.at[wid], src_v)
    pltpu.sync_copy(dst_hbm.at[wid], dst_v)
    pltpu.sync_copy(ones_hbm, ones_v)
    sl = pl.ds(s * RPT, RPT)
    pltpu.sync_copy(z1_hbm.at[sl], dego_sh.at[sl])
    pltpu.sync_copy(z1_hbm.at[sl], degi_sh.at[sl])
    plsc.subcore_barrier()

    @pl.loop(0, CH // GRP)
    def _(j):
        pltpu.sync_copy(ones_v, dego_sh.at[src_v.at[j]], add=True)
        pltpu.sync_copy(ones_v, degi_sh.at[dst_v.at[j]], add=True)

    plsc.subcore_barrier()
    pltpu.sync_copy(dego_sh.at[sl], out_hbm.at[c, 0, sl])
    pltpu.sync_copy(degi_sh.at[sl], out_hbm.at[c, 1, sl])


_deg_call = pl.kernel(
    _deg_body,
    out_type=jax.ShapeDtypeStruct((NC, 2, N_PAD), jnp.float32),
    mesh=_mesh,
    scratch_types=[
        pltpu.VMEM((CH // GRP, GRP * CHUNK), jnp.int32),
        pltpu.VMEM((CH // GRP, GRP * CHUNK), jnp.int32),
        pltpu.VMEM((GRP * CHUNK,), jnp.float32),
        pltpu.VMEM_SHARED((N_PAD,), jnp.float32),
        pltpu.VMEM_SHARED((N_PAD,), jnp.float32),
    ],
    compiler_params=_sc_params,
)


def _layer_body(h_hbm, src_hbm, dst_hbm, out_hbm,
                src_v, dst_v, rows0_v, h_sh, agg_sh):
    c = lax.axis_index("c")
    s = lax.axis_index("s")
    wid = c * NS + s
    pltpu.sync_copy(src_hbm.at[wid], src_v)
    pltpu.sync_copy(dst_hbm.at[wid], dst_v)
    sl = pl.ds(s * RPT, RPT)

    zv = jnp.zeros((32,), jnp.bfloat16)

    @pl.loop(0, RPT)
    def _(r):
        for cb in range(D_H // 32):
            rows0_v[r, pl.ds(cb * 32, 32)] = zv

    pltpu.sync_copy(rows0_v.at[pl.ds(0, RPT)], agg_sh.at[sl])
    pltpu.sync_copy(h_hbm.at[sl], h_sh.at[sl])
    plsc.subcore_barrier()

    @pl.loop(0, CH // GRP)
    def _(i):
        pltpu.sync_copy(h_sh.at[src_v.at[i]], rows0_v)
        pltpu.sync_copy(rows0_v, agg_sh.at[dst_v.at[i]], add=True)

    plsc.subcore_barrier()
    pltpu.sync_copy(agg_sh.at[sl], out_hbm.at[c, sl])


_layer_call = pl.kernel(
    _layer_body,
    out_type=jax.ShapeDtypeStruct((NC, N_PAD, D_H), jnp.bfloat16),
    mesh=_mesh,
    scratch_types=[
        pltpu.VMEM((CH // GRP, GRP * CHUNK), jnp.int32),
        pltpu.VMEM((CH // GRP, GRP * CHUNK), jnp.int32),
        pltpu.VMEM((GRP * CHUNK, D_H), jnp.bfloat16),
        pltpu.VMEM_SHARED((N_PAD, D_H), jnp.bfloat16),
        pltpu.VMEM_SHARED((N_PAD, D_H), jnp.bfloat16),
    ],
    compiler_params=_sc_params,
)


def _prep_body(f_ref, w_ref, degp_ref, xw_ref, ns_ref, nd_ref):
    xw_ref[...] = jnp.dot(f_ref[...], w_ref[...],
                          preferred_element_type=jnp.float32,
                          precision=lax.Precision.HIGHEST)
    dego = degp_ref[0, 0:1, :] + degp_ref[1, 0:1, :]
    degi = degp_ref[0, 1:2, :] + degp_ref[1, 1:2, :]
    ns_ref[...] = jnp.where(dego > 0.0, lax.rsqrt(jnp.maximum(dego, 1.0)), 0.0)
    nd_ref[...] = jnp.where(degi > 0.0, lax.rsqrt(jnp.maximum(degi, 1.0)), 0.0)


_prep_call = pl.pallas_call(
    _prep_body,
    out_shape=(
        jax.ShapeDtypeStruct((N_PAD, D_H), jnp.float32),
        jax.ShapeDtypeStruct((1, N_PAD), jnp.float32),
        jax.ShapeDtypeStruct((1, N_PAD), jnp.float32),
    ),
)


def _scale_body(xw_ref, ns_ref, h_ref):
    h_ref[...] = (xw_ref[...] * ns_ref[...]).astype(jnp.bfloat16)


_scale_call = pl.pallas_call(
    _scale_body,
    out_shape=jax.ShapeDtypeStruct((N_PAD, D_H), jnp.bfloat16),
)


def _combine_body(aggp_ref, nd_ref, ns_ref, b_ref, w_ref, h_ref, out_ref):
    p = aggp_ref[0].astype(jnp.float32) + aggp_ref[1].astype(jnp.float32)
    x = jnp.maximum(p * nd_ref[...] + b_ref[...], 0.0)
    h_ref[...] = (jnp.dot(x, w_ref[...],
                          preferred_element_type=jnp.float32,
                          precision=lax.Precision.HIGHEST)
                  * ns_ref[...]).astype(jnp.bfloat16)
    out_ref[...] = jnp.sum(x[:N, :], axis=0, keepdims=True) * (1.0 / N)


_combine_call = pl.pallas_call(
    _combine_body,
    out_shape=(
        jax.ShapeDtypeStruct((N_PAD, D_H), jnp.bfloat16),
        jax.ShapeDtypeStruct((1, D_H), jnp.float32),
    ),
)


def kernel(features, edge_index, W1, b1, W2, b2, W3, b3):
    src = edge_index[0]
    dst = edge_index[1]
    padv = jnp.full((E_PAD - E,), N, dtype=jnp.int32)
    srcp = jnp.concatenate([src, padv]).reshape(NW, CH // GRP, GRP * CHUNK)
    dstp = jnp.concatenate([dst, padv]).reshape(NW, CH // GRP, GRP * CHUNK)
    featp = jnp.pad(features, ((0, N_PAD - N), (0, 0)))
    ones = jnp.ones((GRP * CHUNK,), jnp.float32)
    z1 = jnp.zeros((N_PAD,), jnp.float32)
    zr = jnp.zeros((N_PAD, D_H), jnp.float32)

    degp = _deg_call(srcp, dstp, ones, z1)
    xw1, ns_row, nd_row = _prep_call(featp, W1, degp)
    ns_col = ns_row.reshape(N_PAD, 1)
    nd_col = nd_row.reshape(N_PAD, 1)

    h = _scale_call(xw1, ns_col)

    w_stack = jnp.stack([W2, W3, W2])
    b_stack = jnp.stack([b1, b2, b3])

    def _body(i, carry):
        h_i, _ = carry
        aggp = _layer_call(h_i, srcp, dstp)
        wn = lax.dynamic_index_in_dim(w_stack, i, keepdims=False)
        bn = lax.dynamic_index_in_dim(b_stack, i, keepdims=False).reshape(1, D_H)
        h_next, out_row = _combine_call(aggp, nd_col, ns_col, bn, wn)
        return (h_next, out_row)

    _, out = lax.fori_loop(
        0, 3, _body, (h, jnp.zeros((1, D_H), jnp.float32)))
    return out.reshape(D_H)


# speedup vs baseline: 2.8934x; 1.0383x over previous
import jax
import jax.numpy as jnp
from jax import lax
from jax.experimental import pallas as pl
from jax.experimental.pallas import tpu as pltpu
from jax.experimental.pallas import tpu_sc as plsc

N = 10000
E = 320000
D_IN = 128
D_H = 64

NC = 2
NS = 16
NW = NC * NS
CHUNK = 128
CH = 80
GRP = 8
E_PAD = NW * CH * CHUNK
N_PAD = 10240
RPT = N_PAD // NS

_mesh = plsc.VectorSubcoreMesh(core_axis_name="c", subcore_axis_name="s")
_sc_params = pltpu.CompilerParams(use_tc_tiling_on_sc=False)


def _deg_body(src_hbm, dst_hbm, ones_hbm, z1_hbm, out_hbm,
              src_v, dst_v, ones_v, dego_sh, degi_sh):
    c = lax.axis_index("c")
    s = lax.axis_index("s")
    wid = c * NS + s
    pltpu.sync_copy(src_hbm.at[wid], src_v)
    pltpu.sync_copy(dst_hbm.at[wid], dst_v)
    pltpu.sync_copy(ones_hbm, ones_v)
    sl = pl.ds(s * RPT, RPT)
    pltpu.sync_copy(z1_hbm.at[sl], dego_sh.at[sl])
    pltpu.sync_copy(z1_hbm.at[sl], degi_sh.at[sl])
    plsc.subcore_barrier()

    @pl.loop(0, CH // GRP)
    def _(j):
        pltpu.sync_copy(ones_v, dego_sh.at[src_v.at[j]], add=True)
        pltpu.sync_copy(ones_v, degi_sh.at[dst_v.at[j]], add=True)

    plsc.subcore_barrier()
    pltpu.sync_copy(dego_sh.at[sl], out_hbm.at[c, 0, sl])
    pltpu.sync_copy(degi_sh.at[sl], out_hbm.at[c, 1, sl])


_deg_call = pl.kernel(
    _deg_body,
    out_type=jax.ShapeDtypeStruct((NC, 2, N_PAD), jnp.float32),
    mesh=_mesh,
    scratch_types=[
        pltpu.VMEM((CH // GRP, GRP * CHUNK), jnp.int32),
        pltpu.VMEM((CH // GRP, GRP * CHUNK), jnp.int32),
        pltpu.VMEM((GRP * CHUNK,), jnp.float32),
        pltpu.VMEM_SHARED((N_PAD,), jnp.float32),
        pltpu.VMEM_SHARED((N_PAD,), jnp.float32),
    ],
    compiler_params=_sc_params,
)


def _layer_body(h_hbm, src_hbm, dst_hbm, out_hbm,
                src_v, dst_v, rows0_v, h_sh, agg_sh):
    c = lax.axis_index("c")
    s = lax.axis_index("s")
    wid = c * NS + s
    pltpu.sync_copy(src_hbm.at[wid], src_v)
    pltpu.sync_copy(dst_hbm.at[wid], dst_v)
    sl = pl.ds(s * RPT, RPT)

    zv = jnp.zeros((32,), jnp.bfloat16)

    @pl.loop(0, RPT)
    def _(r):
        for cb in range(D_H // 32):
            rows0_v[r, pl.ds(cb * 32, 32)] = zv

    pltpu.sync_copy(rows0_v.at[pl.ds(0, RPT)], agg_sh.at[sl])
    pltpu.sync_copy(h_hbm.at[sl], h_sh.at[sl])
    plsc.subcore_barrier()

    @pl.loop(0, CH // GRP)
    def _(i):
        pltpu.sync_copy(h_sh.at[src_v.at[i]], rows0_v)
        pltpu.sync_copy(rows0_v, agg_sh.at[dst_v.at[i]], add=True)

    plsc.subcore_barrier()
    pltpu.sync_copy(agg_sh.at[sl], out_hbm.at[c, sl])


_layer_call = pl.kernel(
    _layer_body,
    out_type=jax.ShapeDtypeStruct((NC, N_PAD, D_H), jnp.bfloat16),
    mesh=_mesh,
    scratch_types=[
        pltpu.VMEM((CH // GRP, GRP * CHUNK), jnp.int32),
        pltpu.VMEM((CH // GRP, GRP * CHUNK), jnp.int32),
        pltpu.VMEM((GRP * CHUNK, D_H), jnp.bfloat16),
        pltpu.VMEM_SHARED((N_PAD, D_H), jnp.bfloat16),
        pltpu.VMEM_SHARED((N_PAD, D_H), jnp.bfloat16),
    ],
    compiler_params=_sc_params,
)


def _mm_body(f_ref, w_ref, xw_ref):
    xw_ref[...] = jnp.dot(f_ref[...], w_ref[...],
                          preferred_element_type=jnp.float32,
                          precision=lax.Precision.HIGHEST)


_mm_call = pl.pallas_call(
    _mm_body,
    out_shape=jax.ShapeDtypeStruct((N_PAD, D_H), jnp.float32),
)


def _prep_body(degp_ref, xw_ref, h_ref, ns_ref, nd_ref):
    dego = degp_ref[0, 0:1, :] + degp_ref[1, 0:1, :]
    degi = degp_ref[0, 1:2, :] + degp_ref[1, 1:2, :]
    ns_row = jnp.where(dego > 0.0, lax.rsqrt(jnp.maximum(dego, 1.0)), 0.0)
    nd_row = jnp.where(degi > 0.0, lax.rsqrt(jnp.maximum(degi, 1.0)), 0.0)
    ns_col = jnp.transpose(ns_row, (1, 0))
    nd_col = jnp.transpose(nd_row, (1, 0))
    ns_ref[...] = ns_col
    nd_ref[...] = nd_col
    h_ref[...] = (xw_ref[...] * ns_col).astype(jnp.bfloat16)


_prep_call = pl.pallas_call(
    _prep_body,
    out_shape=(
        jax.ShapeDtypeStruct((N_PAD, D_H), jnp.bfloat16),
        jax.ShapeDtypeStruct((N_PAD, 1), jnp.float32),
        jax.ShapeDtypeStruct((N_PAD, 1), jnp.float32),
    ),
)


def _combine_body(aggp_ref, nd_ref, ns_ref, b_ref, w_ref, h_ref, out_ref):
    p = aggp_ref[0].astype(jnp.float32) + aggp_ref[1].astype(jnp.float32)
    x = jnp.maximum(p * nd_ref[...] + b_ref[...], 0.0)
    h_ref[...] = (jnp.dot(x, w_ref[...],
                          preferred_element_type=jnp.float32,
                          precision=lax.Precision.HIGHEST)
                  * ns_ref[...]).astype(jnp.bfloat16)
    out_ref[...] = jnp.sum(x[:N, :], axis=0, keepdims=True) * (1.0 / N)


_combine_call = pl.pallas_call(
    _combine_body,
    out_shape=(
        jax.ShapeDtypeStruct((N_PAD, D_H), jnp.bfloat16),
        jax.ShapeDtypeStruct((1, D_H), jnp.float32),
    ),
)


def kernel(features, edge_index, W1, b1, W2, b2, W3, b3):
    src = edge_index[0]
    dst = edge_index[1]
    padv = jnp.full((E_PAD - E,), N, dtype=jnp.int32)
    srcp = jnp.concatenate([src, padv]).reshape(NW, CH // GRP, GRP * CHUNK)
    dstp = jnp.concatenate([dst, padv]).reshape(NW, CH // GRP, GRP * CHUNK)
    featp = jnp.pad(features, ((0, N_PAD - N), (0, 0)))
    ones = jnp.ones((GRP * CHUNK,), jnp.float32)
    z1 = jnp.zeros((N_PAD,), jnp.float32)
    zr = jnp.zeros((N_PAD, D_H), jnp.float32)

    degp = _deg_call(srcp, dstp, ones, z1)
    xw1 = _mm_call(featp, W1)
    h, ns_col, nd_col = _prep_call(degp, xw1)

    w_stack = jnp.stack([W2, W3, W2])
    b_stack = jnp.stack([b1, b2, b3])

    def _body(i, carry):
        h_i, _ = carry
        aggp = _layer_call(h_i, srcp, dstp)
        wn = lax.dynamic_index_in_dim(w_stack, i, keepdims=False)
        bn = lax.dynamic_index_in_dim(b_stack, i, keepdims=False).reshape(1, D_H)
        h_next, out_row = _combine_call(aggp, nd_col, ns_col, bn, wn)
        return (h_next, out_row)

    _, out = lax.fori_loop(
        0, 3, _body, (h, jnp.zeros((1, D_H), jnp.float32)))
    return out.reshape(D_H)


# speedup vs baseline: 2.8993x; 1.0020x over previous
import jax
import jax.numpy as jnp
from jax import lax
from jax.experimental import pallas as pl
from jax.experimental.pallas import tpu as pltpu
from jax.experimental.pallas import tpu_sc as plsc

N = 10000
E = 320000
D_IN = 128
D_H = 64

NC = 2
NS = 16
NW = NC * NS
CHUNK = 128
CH = 80
GRP = 8
E_PAD = NW * CH * CHUNK
N_PAD = 10240
RPT = N_PAD // NS

_mesh = plsc.VectorSubcoreMesh(core_axis_name="c", subcore_axis_name="s")
_sc_params = pltpu.CompilerParams(use_tc_tiling_on_sc=False)


def _deg_body(src_hbm, dst_hbm, ones_hbm, z1_hbm, out_hbm,
              src_v, dst_v, ones_v, dego_sh, degi_sh):
    c = lax.axis_index("c")
    s = lax.axis_index("s")
    wid = c * NS + s
    pltpu.sync_copy(src_hbm.at[wid], src_v)
    pltpu.sync_copy(dst_hbm.at[wid], dst_v)
    pltpu.sync_copy(ones_hbm, ones_v)
    sl = pl.ds(s * RPT, RPT)
    pltpu.sync_copy(z1_hbm.at[sl], dego_sh.at[sl])
    pltpu.sync_copy(z1_hbm.at[sl], degi_sh.at[sl])
    plsc.subcore_barrier()

    @pl.loop(0, CH // GRP)
    def _(j):
        pltpu.sync_copy(ones_v, dego_sh.at[src_v.at[j]], add=True)
        pltpu.sync_copy(ones_v, degi_sh.at[dst_v.at[j]], add=True)

    plsc.subcore_barrier()
    pltpu.sync_copy(dego_sh.at[sl], out_hbm.at[c, 0, sl])
    pltpu.sync_copy(degi_sh.at[sl], out_hbm.at[c, 1, sl])


_deg_call = pl.kernel(
    _deg_body,
    out_type=jax.ShapeDtypeStruct((NC, 2, N_PAD), jnp.float32),
    mesh=_mesh,
    scratch_types=[
        pltpu.VMEM((CH // GRP, GRP * CHUNK), jnp.int32),
        pltpu.VMEM((CH // GRP, GRP * CHUNK), jnp.int32),
        pltpu.VMEM((GRP * CHUNK,), jnp.float32),
        pltpu.VMEM_SHARED((N_PAD,), jnp.float32),
        pltpu.VMEM_SHARED((N_PAD,), jnp.float32),
    ],
    compiler_params=_sc_params,
)


def _layer_body(h_hbm, src_hbm, dst_hbm, out_hbm,
                src_v, dst_v, rows0_v, h_sh, agg_sh):
    c = lax.axis_index("c")
    s = lax.axis_index("s")
    wid = c * NS + s
    pltpu.sync_copy(src_hbm.at[wid], src_v)
    pltpu.sync_copy(dst_hbm.at[wid], dst_v)
    sl = pl.ds(s * RPT, RPT)

    zv = jnp.zeros((32,), jnp.bfloat16)

    @pl.loop(0, RPT)
    def _(r):
        for cb in range(D_H // 32):
            rows0_v[r, pl.ds(cb * 32, 32)] = zv

    pltpu.sync_copy(rows0_v.at[pl.ds(0, RPT)], agg_sh.at[sl])
    pltpu.sync_copy(h_hbm.at[sl], h_sh.at[sl])
    plsc.subcore_barrier()

    @pl.loop(0, CH // GRP)
    def _(i):
        pltpu.sync_copy(h_sh.at[src_v.at[i]], rows0_v)
        pltpu.sync_copy(rows0_v, agg_sh.at[dst_v.at[i]], add=True)

    plsc.subcore_barrier()
    pltpu.sync_copy(agg_sh.at[sl], out_hbm.at[c, sl])


_layer_call = pl.kernel(
    _layer_body,
    out_type=jax.ShapeDtypeStruct((NC, N_PAD, D_H), jnp.bfloat16),
    mesh=_mesh,
    scratch_types=[
        pltpu.VMEM((CH // GRP, GRP * CHUNK), jnp.int32),
        pltpu.VMEM((CH // GRP, GRP * CHUNK), jnp.int32),
        pltpu.VMEM((GRP * CHUNK, D_H), jnp.bfloat16),
        pltpu.VMEM_SHARED((N_PAD, D_H), jnp.bfloat16),
        pltpu.VMEM_SHARED((N_PAD, D_H), jnp.bfloat16),
    ],
    compiler_params=_sc_params,
)


def _mm_body(f_ref, w_ref, xw_ref):
    xw_ref[...] = jnp.dot(f_ref[...], w_ref[...],
                          preferred_element_type=jnp.float32,
                          precision=lax.Precision.HIGHEST)


_mm_call = pl.pallas_call(
    _mm_body,
    out_shape=jax.ShapeDtypeStruct((N_PAD, D_H), jnp.float32),
)


def _prep_body(degp_ref, xw_ref, h_ref, ns_ref, nd_ref):
    dego = degp_ref[0, 0:1, :] + degp_ref[1, 0:1, :]
    degi = degp_ref[0, 1:2, :] + degp_ref[1, 1:2, :]
    ns_row = jnp.where(dego > 0.0, lax.rsqrt(jnp.maximum(dego, 1.0)), 0.0)
    nd_row = jnp.where(degi > 0.0, lax.rsqrt(jnp.maximum(degi, 1.0)), 0.0)
    ns_col = jnp.transpose(ns_row, (1, 0))
    nd_col = jnp.transpose(nd_row, (1, 0))
    ns_ref[...] = ns_col
    nd_ref[...] = nd_col
    h_ref[...] = (xw_ref[...] * ns_col).astype(jnp.bfloat16)


_prep_call = pl.pallas_call(
    _prep_body,
    out_shape=(
        jax.ShapeDtypeStruct((N_PAD, D_H), jnp.bfloat16),
        jax.ShapeDtypeStruct((N_PAD, 1), jnp.float32),
        jax.ShapeDtypeStruct((N_PAD, 1), jnp.float32),
    ),
)


def _combine_body(aggp_ref, nd_ref, ns_ref, b_ref, w_ref, h_ref):
    p = aggp_ref[0].astype(jnp.float32) + aggp_ref[1].astype(jnp.float32)
    x = jnp.maximum(p * nd_ref[...] + b_ref[...], 0.0)
    h_ref[...] = (jnp.dot(x, w_ref[...],
                          preferred_element_type=jnp.float32,
                          precision=lax.Precision.HIGHEST)
                  * ns_ref[...]).astype(jnp.bfloat16)


_combine_call = pl.pallas_call(
    _combine_body,
    out_shape=jax.ShapeDtypeStruct((N_PAD, D_H), jnp.bfloat16),
)


def _final_body(aggp_ref, nd_ref, b_ref, out_ref):
    p = aggp_ref[0].astype(jnp.float32) + aggp_ref[1].astype(jnp.float32)
    x = jnp.maximum(p * nd_ref[...] + b_ref[...], 0.0)
    out_ref[...] = jnp.sum(x[:N, :], axis=0, keepdims=True) * (1.0 / N)


_final_call = pl.pallas_call(
    _final_body,
    out_shape=jax.ShapeDtypeStruct((1, D_H), jnp.float32),
)


def kernel(features, edge_index, W1, b1, W2, b2, W3, b3):
    src = edge_index[0]
    dst = edge_index[1]
    padv = jnp.full((E_PAD - E,), N, dtype=jnp.int32)
    srcp = jnp.concatenate([src, padv]).reshape(NW, CH // GRP, GRP * CHUNK)
    dstp = jnp.concatenate([dst, padv]).reshape(NW, CH // GRP, GRP * CHUNK)
    featp = jnp.pad(features, ((0, N_PAD - N), (0, 0)))
    ones = jnp.ones((GRP * CHUNK,), jnp.float32)
    z1 = jnp.zeros((N_PAD,), jnp.float32)
    zr = jnp.zeros((N_PAD, D_H), jnp.float32)

    degp = _deg_call(srcp, dstp, ones, z1)
    xw1 = _mm_call(featp, W1)
    h, ns_col, nd_col = _prep_call(degp, xw1)

    agg1 = _layer_call(h, srcp, dstp)
    h2 = _combine_call(agg1, nd_col, ns_col, b1.reshape(1, D_H), W2)
    agg2 = _layer_call(h2, srcp, dstp)
    h3 = _combine_call(agg2, nd_col, ns_col, b2.reshape(1, D_H), W3)
    agg3 = _layer_call(h3, srcp, dstp)
    out = _final_call(agg3, nd_col, b3.reshape(1, D_H))
    return out.reshape(D_H)


# speedup vs baseline: 3.1999x; 1.1037x over previous
import jax
import jax.numpy as jnp
from jax import lax
from jax.experimental import pallas as pl
from jax.experimental.pallas import tpu as pltpu
from jax.experimental.pallas import tpu_sc as plsc

N = 10000
E = 320000
D_IN = 128
D_H = 64

NC = 2
NS = 16
NW = NC * NS
CHUNK = 128
CH = 80
GRP = 8
E_PAD = NW * CH * CHUNK
N_PAD = 10240
RPT = N_PAD // NS

_mesh = plsc.VectorSubcoreMesh(core_axis_name="c", subcore_axis_name="s")
_sc_params = pltpu.CompilerParams(use_tc_tiling_on_sc=False)


def _deg_body(src_hbm, dst_hbm, ones_hbm, z1_hbm, out_hbm,
              src_v, dst_v, ones_v, dego_sh, degi_sh):
    c = lax.axis_index("c")
    s = lax.axis_index("s")
    wid = c * NS + s
    pltpu.sync_copy(src_hbm.at[wid], src_v)
    pltpu.sync_copy(dst_hbm.at[wid], dst_v)
    pltpu.sync_copy(ones_hbm, ones_v)
    sl = pl.ds(s * RPT, RPT)
    pltpu.sync_copy(z1_hbm.at[sl], dego_sh.at[sl])
    pltpu.sync_copy(z1_hbm.at[sl], degi_sh.at[sl])
    plsc.subcore_barrier()

    @pl.loop(0, CH // GRP)
    def _(j):
        pltpu.sync_copy(ones_v, dego_sh.at[src_v.at[j]], add=True)
        pltpu.sync_copy(ones_v, degi_sh.at[dst_v.at[j]], add=True)

    plsc.subcore_barrier()
    pltpu.sync_copy(dego_sh.at[sl], out_hbm.at[c, 0, sl])
    pltpu.sync_copy(degi_sh.at[sl], out_hbm.at[c, 1, sl])


_deg_call = pl.kernel(
    _deg_body,
    out_type=jax.ShapeDtypeStruct((NC, 2, N_PAD), jnp.float32),
    mesh=_mesh,
    scratch_types=[
        pltpu.VMEM((CH // GRP, GRP * CHUNK), jnp.int32),
        pltpu.VMEM((CH // GRP, GRP * CHUNK), jnp.int32),
        pltpu.VMEM((GRP * CHUNK,), jnp.float32),
        pltpu.VMEM_SHARED((N_PAD,), jnp.float32),
        pltpu.VMEM_SHARED((N_PAD,), jnp.float32),
    ],
    compiler_params=_sc_params,
)


def _layer_body(h_hbm, src_hbm, dst_hbm, out_hbm,
                src_v, dst_v, rows0_v, rows1_v, h_sh, agg_sh, sem0, sem1):
    c = lax.axis_index("c")
    s = lax.axis_index("s")
    wid = c * NS + s
    pltpu.sync_copy(src_hbm.at[wid], src_v)
    pltpu.sync_copy(dst_hbm.at[wid], dst_v)
    sl = pl.ds(s * RPT, RPT)

    zv = jnp.zeros((32,), jnp.bfloat16)

    @pl.loop(0, RPT)
    def _(r):
        for cb in range(D_H // 32):
            rows0_v[r, pl.ds(cb * 32, 32)] = zv

    pltpu.sync_copy(rows0_v.at[pl.ds(0, RPT)], agg_sh.at[sl])
    pltpu.sync_copy(h_hbm.at[sl], h_sh.at[sl])
    plsc.subcore_barrier()

    pltpu.async_copy(h_sh.at[src_v.at[0]], rows0_v, sem0)
    pltpu.async_copy(h_sh.at[src_v.at[1]], rows1_v, sem1)

    @pl.loop(0, CH // GRP // 2 - 1)
    def _(k):
        j = 2 * k
        pltpu.make_async_copy(h_sh.at[src_v.at[0]], rows0_v, sem0).wait()
        pltpu.sync_copy(rows0_v, agg_sh.at[dst_v.at[j]], add=True)
        pltpu.async_copy(h_sh.at[src_v.at[j + 2]], rows0_v, sem0)
        pltpu.make_async_copy(h_sh.at[src_v.at[1]], rows1_v, sem1).wait()
        pltpu.sync_copy(rows1_v, agg_sh.at[dst_v.at[j + 1]], add=True)
        pltpu.async_copy(h_sh.at[src_v.at[j + 3]], rows1_v, sem1)

    pltpu.make_async_copy(h_sh.at[src_v.at[0]], rows0_v, sem0).wait()
    pltpu.sync_copy(rows0_v, agg_sh.at[dst_v.at[CH // GRP - 2]], add=True)
    pltpu.make_async_copy(h_sh.at[src_v.at[1]], rows1_v, sem1).wait()
    pltpu.sync_copy(rows1_v, agg_sh.at[dst_v.at[CH // GRP - 1]], add=True)

    plsc.subcore_barrier()
    pltpu.sync_copy(agg_sh.at[sl], out_hbm.at[c, sl])


_layer_call = pl.kernel(
    _layer_body,
    out_type=jax.ShapeDtypeStruct((NC, N_PAD, D_H), jnp.bfloat16),
    mesh=_mesh,
    scratch_types=[
        pltpu.VMEM((CH // GRP, GRP * CHUNK), jnp.int32),
        pltpu.VMEM((CH // GRP, GRP * CHUNK), jnp.int32),
        pltpu.VMEM((GRP * CHUNK, D_H), jnp.bfloat16),
        pltpu.VMEM((GRP * CHUNK, D_H), jnp.bfloat16),
        pltpu.VMEM_SHARED((N_PAD, D_H), jnp.bfloat16),
        pltpu.VMEM_SHARED((N_PAD, D_H), jnp.bfloat16),
        pltpu.SemaphoreType.DMA,
        pltpu.SemaphoreType.DMA,
    ],
    compiler_params=_sc_params,
)


def _mm_body(f_ref, w_ref, xw_ref):
    xw_ref[...] = jnp.dot(f_ref[...], w_ref[...],
                          preferred_element_type=jnp.float32,
                          precision=lax.Precision.HIGHEST)


_mm_call = pl.pallas_call(
    _mm_body,
    out_shape=jax.ShapeDtypeStruct((N_PAD, D_H), jnp.float32),
)


def _prep_body(degp_ref, xw_ref, h_ref, ns_ref, nd_ref):
    dego = degp_ref[0, 0:1, :] + degp_ref[1, 0:1, :]
    degi = degp_ref[0, 1:2, :] + degp_ref[1, 1:2, :]
    ns_row = jnp.where(dego > 0.0, lax.rsqrt(jnp.maximum(dego, 1.0)), 0.0)
    nd_row = jnp.where(degi > 0.0, lax.rsqrt(jnp.maximum(degi, 1.0)), 0.0)
    ns_col = jnp.transpose(ns_row, (1, 0))
    nd_col = jnp.transpose(nd_row, (1, 0))
    ns_ref[...] = ns_col
    nd_ref[...] = nd_col
    h_ref[...] = (xw_ref[...] * ns_col).astype(jnp.bfloat16)


_prep_call = pl.pallas_call(
    _prep_body,
    out_shape=(
        jax.ShapeDtypeStruct((N_PAD, D_H), jnp.bfloat16),
        jax.ShapeDtypeStruct((N_PAD, 1), jnp.float32),
        jax.ShapeDtypeStruct((N_PAD, 1), jnp.float32),
    ),
)


def _combine_body(aggp_ref, nd_ref, ns_ref, b_ref, w_ref, h_ref):
    p = aggp_ref[0].astype(jnp.float32) + aggp_ref[1].astype(jnp.float32)
    x = jnp.maximum(p * nd_ref[...] + b_ref[...], 0.0)
    h_ref[...] = (jnp.dot(x, w_ref[...],
                          preferred_element_type=jnp.float32,
                          precision=lax.Precision.HIGHEST)
                  * ns_ref[...]).astype(jnp.bfloat16)


_combine_call = pl.pallas_call(
    _combine_body,
    out_shape=jax.ShapeDtypeStruct((N_PAD, D_H), jnp.bfloat16),
)


def _final_body(aggp_ref, nd_ref, b_ref, out_ref):
    p = aggp_ref[0].astype(jnp.float32) + aggp_ref[1].astype(jnp.float32)
    x = jnp.maximum(p * nd_ref[...] + b_ref[...], 0.0)
    out_ref[...] = jnp.sum(x[:N, :], axis=0, keepdims=True) * (1.0 / N)


_final_call = pl.pallas_call(
    _final_body,
    out_shape=jax.ShapeDtypeStruct((1, D_H), jnp.float32),
)


def kernel(features, edge_index, W1, b1, W2, b2, W3, b3):
    src = edge_index[0]
    dst = edge_index[1]
    padv = jnp.full((E_PAD - E,), N, dtype=jnp.int32)
    srcp = jnp.concatenate([src, padv]).reshape(NW, CH // GRP, GRP * CHUNK)
    dstp = jnp.concatenate([dst, padv]).reshape(NW, CH // GRP, GRP * CHUNK)
    featp = jnp.pad(features, ((0, N_PAD - N), (0, 0)))
    ones = jnp.ones((GRP * CHUNK,), jnp.float32)
    z1 = jnp.zeros((N_PAD,), jnp.float32)
    zr = jnp.zeros((N_PAD, D_H), jnp.float32)

    degp = _deg_call(srcp, dstp, ones, z1)
    xw1 = _mm_call(featp, W1)
    h, ns_col, nd_col = _prep_call(degp, xw1)

    agg1 = _layer_call(h, srcp, dstp)
    h2 = _combine_call(agg1, nd_col, ns_col, b1.reshape(1, D_H), W2)
    agg2 = _layer_call(h2, srcp, dstp)
    h3 = _combine_call(agg2, nd_col, ns_col, b2.reshape(1, D_H), W3)
    agg3 = _layer_call(h3, srcp, dstp)
    out = _final_call(agg3, nd_col, b3.reshape(1, D_H))
    return out.reshape(D_H)


# speedup vs baseline: 3.2437x; 1.0137x over previous
import jax
import jax.numpy as jnp
from jax import lax
from jax.experimental import pallas as pl
from jax.experimental.pallas import tpu as pltpu
from jax.experimental.pallas import tpu_sc as plsc

N = 10000
E = 320000
D_IN = 128
D_H = 64

NC = 2
NS = 16
NW = NC * NS
CHUNK = 128
CH = 80
GRP = 8
DGRP = 16
E_PAD = NW * CH * CHUNK
N_PAD = 10240
RPT = N_PAD // NS

_mesh = plsc.VectorSubcoreMesh(core_axis_name="c", subcore_axis_name="s")
_sc_params = pltpu.CompilerParams(use_tc_tiling_on_sc=False)


def _deg_body(src_hbm, dst_hbm, ones_hbm, z1_hbm, out_hbm,
              src_v, dst_v, ones_v, dego_sh, degi_sh, sem0, sem1):
    c = lax.axis_index("c")
    s = lax.axis_index("s")
    wid = c * NS + s
    pltpu.sync_copy(src_hbm.at[wid], src_v)
    pltpu.sync_copy(dst_hbm.at[wid], dst_v)
    pltpu.sync_copy(ones_hbm, ones_v)
    sl = pl.ds(s * RPT, RPT)
    pltpu.sync_copy(z1_hbm.at[sl], dego_sh.at[sl])
    pltpu.sync_copy(z1_hbm.at[sl], degi_sh.at[sl])
    plsc.subcore_barrier()

    @pl.loop(0, CH // DGRP)
    def _(j):
        pltpu.async_copy(ones_v, dego_sh.at[src_v.at[j]], sem0, add=True)
        pltpu.async_copy(ones_v, degi_sh.at[dst_v.at[j]], sem1, add=True)

    @pl.loop(0, CH // DGRP)
    def _(j):
        pltpu.make_async_copy(ones_v, dego_sh.at[src_v.at[0]], sem0).wait()
        pltpu.make_async_copy(ones_v, degi_sh.at[dst_v.at[0]], sem1).wait()

    plsc.subcore_barrier()
    pltpu.sync_copy(dego_sh.at[sl], out_hbm.at[c, 0, sl])
    pltpu.sync_copy(degi_sh.at[sl], out_hbm.at[c, 1, sl])


_deg_call = pl.kernel(
    _deg_body,
    out_type=jax.ShapeDtypeStruct((NC, 2, N_PAD), jnp.float32),
    mesh=_mesh,
    scratch_types=[
        pltpu.VMEM((CH // DGRP, DGRP * CHUNK), jnp.int32),
        pltpu.VMEM((CH // DGRP, DGRP * CHUNK), jnp.int32),
        pltpu.VMEM((DGRP * CHUNK,), jnp.float32),
        pltpu.VMEM_SHARED((N_PAD,), jnp.float32),
        pltpu.VMEM_SHARED((N_PAD,), jnp.float32),
        pltpu.SemaphoreType.DMA,
        pltpu.SemaphoreType.DMA,
    ],
    compiler_params=_sc_params,
)


def _layer_body(h_hbm, src_hbm, dst_hbm, out_hbm,
                src_v, dst_v, rows0_v, rows1_v, h_sh, agg_sh, sem0, sem1):
    c = lax.axis_index("c")
    s = lax.axis_index("s")
    wid = c * NS + s
    pltpu.sync_copy(src_hbm.at[wid], src_v)
    pltpu.sync_copy(dst_hbm.at[wid], dst_v)
    sl = pl.ds(s * RPT, RPT)

    zv = jnp.zeros((32,), jnp.bfloat16)

    @pl.loop(0, RPT)
    def _(r):
        for cb in range(D_H // 32):
            rows0_v[r, pl.ds(cb * 32, 32)] = zv

    pltpu.sync_copy(rows0_v.at[pl.ds(0, RPT)], agg_sh.at[sl])
    pltpu.sync_copy(h_hbm.at[sl], h_sh.at[sl])
    plsc.subcore_barrier()

    pltpu.async_copy(h_sh.at[src_v.at[0]], rows0_v, sem0)
    pltpu.async_copy(h_sh.at[src_v.at[1]], rows1_v, sem1)

    @pl.loop(0, CH // GRP // 2 - 1)
    def _(k):
        j = 2 * k
        pltpu.make_async_copy(h_sh.at[src_v.at[0]], rows0_v, sem0).wait()
        pltpu.sync_copy(rows0_v, agg_sh.at[dst_v.at[j]], add=True)
        pltpu.async_copy(h_sh.at[src_v.at[j + 2]], rows0_v, sem0)
        pltpu.make_async_copy(h_sh.at[src_v.at[1]], rows1_v, sem1).wait()
        pltpu.sync_copy(rows1_v, agg_sh.at[dst_v.at[j + 1]], add=True)
        pltpu.async_copy(h_sh.at[src_v.at[j + 3]], rows1_v, sem1)

    pltpu.make_async_copy(h_sh.at[src_v.at[0]], rows0_v, sem0).wait()
    pltpu.sync_copy(rows0_v, agg_sh.at[dst_v.at[CH // GRP - 2]], add=True)
    pltpu.make_async_copy(h_sh.at[src_v.at[1]], rows1_v, sem1).wait()
    pltpu.sync_copy(rows1_v, agg_sh.at[dst_v.at[CH // GRP - 1]], add=True)

    plsc.subcore_barrier()
    pltpu.sync_copy(agg_sh.at[sl], out_hbm.at[c, sl])


_layer_call = pl.kernel(
    _layer_body,
    out_type=jax.ShapeDtypeStruct((NC, N_PAD, D_H), jnp.bfloat16),
    mesh=_mesh,
    scratch_types=[
        pltpu.VMEM((CH // GRP, GRP * CHUNK), jnp.int32),
        pltpu.VMEM((CH // GRP, GRP * CHUNK), jnp.int32),
        pltpu.VMEM((GRP * CHUNK, D_H), jnp.bfloat16),
        pltpu.VMEM((GRP * CHUNK, D_H), jnp.bfloat16),
        pltpu.VMEM_SHARED((N_PAD, D_H), jnp.bfloat16),
        pltpu.VMEM_SHARED((N_PAD, D_H), jnp.bfloat16),
        pltpu.SemaphoreType.DMA,
        pltpu.SemaphoreType.DMA,
    ],
    compiler_params=_sc_params,
)


def _mm_body(f_ref, w_ref, xw_ref):
    xw_ref[...] = jnp.dot(f_ref[...], w_ref[...],
                          preferred_element_type=jnp.float32,
                          precision=lax.Precision.HIGHEST)


_mm_call = pl.pallas_call(
    _mm_body,
    out_shape=jax.ShapeDtypeStruct((N_PAD, D_H), jnp.float32),
)


def _prep_body(degp_ref, xw_ref, h_ref, ns_ref, nd_ref):
    dego = degp_ref[0, 0:1, :] + degp_ref[1, 0:1, :]
    degi = degp_ref[0, 1:2, :] + degp_ref[1, 1:2, :]
    ns_row = jnp.where(dego > 0.0, lax.rsqrt(jnp.maximum(dego, 1.0)), 0.0)
    nd_row = jnp.where(degi > 0.0, lax.rsqrt(jnp.maximum(degi, 1.0)), 0.0)
    ns_col = jnp.transpose(ns_row, (1, 0))
    nd_col = jnp.transpose(nd_row, (1, 0))
    ns_ref[...] = ns_col
    nd_ref[...] = nd_col
    h_ref[...] = (xw_ref[...] * ns_col).astype(jnp.bfloat16)


_prep_call = pl.pallas_call(
    _prep_body,
    out_shape=(
        jax.ShapeDtypeStruct((N_PAD, D_H), jnp.bfloat16),
        jax.ShapeDtypeStruct((N_PAD, 1), jnp.float32),
        jax.ShapeDtypeStruct((N_PAD, 1), jnp.float32),
    ),
)


def _combine_body(aggp_ref, nd_ref, ns_ref, b_ref, w_ref, h_ref):
    p = aggp_ref[0].astype(jnp.float32) + aggp_ref[1].astype(jnp.float32)
    x = jnp.maximum(p * nd_ref[...] + b_ref[...], 0.0)
    h_ref[...] = (jnp.dot(x, w_ref[...],
                          preferred_element_type=jnp.float32,
                          precision=lax.Precision.HIGHEST)
                  * ns_ref[...]).astype(jnp.bfloat16)


_combine_call = pl.pallas_call(
    _combine_body,
    out_shape=jax.ShapeDtypeStruct((N_PAD, D_H), jnp.bfloat16),
)


def _final_body(aggp_ref, nd_ref, b_ref, out_ref):
    p = aggp_ref[0].astype(jnp.float32) + aggp_ref[1].astype(jnp.float32)
    x = jnp.maximum(p * nd_ref[...] + b_ref[...], 0.0)
    out_ref[...] = jnp.sum(x[:N, :], axis=0, keepdims=True) * (1.0 / N)


_final_call = pl.pallas_call(
    _final_body,
    out_shape=jax.ShapeDtypeStruct((1, D_H), jnp.float32),
)


def kernel(features, edge_index, W1, b1, W2, b2, W3, b3):
    src = edge_index[0]
    dst = edge_index[1]
    padv = jnp.full((E_PAD - E,), N, dtype=jnp.int32)
    srcp = jnp.concatenate([src, padv]).reshape(NW, CH // GRP, GRP * CHUNK)
    dstp = jnp.concatenate([dst, padv]).reshape(NW, CH // GRP, GRP * CHUNK)
    featp = jnp.pad(features, ((0, N_PAD - N), (0, 0)))
    ones = jnp.ones((DGRP * CHUNK,), jnp.float32)
    z1 = jnp.zeros((N_PAD,), jnp.float32)
    srcd = srcp.reshape(NW, CH // DGRP, DGRP * CHUNK)
    dstd = dstp.reshape(NW, CH // DGRP, DGRP * CHUNK)

    degp = _deg_call(srcd, dstd, ones, z1)
    xw1 = _mm_call(featp, W1)
    h, ns_col, nd_col = _prep_call(degp, xw1)

    agg1 = _layer_call(h, srcp, dstp)
    h2 = _combine_call(agg1, nd_col, ns_col, b1.reshape(1, D_H), W2)
    agg2 = _layer_call(h2, srcp, dstp)
    h3 = _combine_call(agg2, nd_col, ns_col, b2.reshape(1, D_H), W3)
    agg3 = _layer_call(h3, srcp, dstp)
    out = _final_call(agg3, nd_col, b3.reshape(1, D_H))
    return out.reshape(D_H)


# speedup vs baseline: 3.3886x; 1.0447x over previous
import jax
import jax.numpy as jnp
from jax import lax
from jax.experimental import pallas as pl
from jax.experimental.pallas import tpu as pltpu
from jax.experimental.pallas import tpu_sc as plsc

N = 10000
E = 320000
D_IN = 128
D_H = 64

NC = 2
NS = 16
NW = NC * NS
CHUNK = 128
CH = 80
GRP = 8
DGRP = 16
E_PAD = NW * CH * CHUNK
N_PAD = 10240
RPT = N_PAD // NS

_mesh = plsc.VectorSubcoreMesh(core_axis_name="c", subcore_axis_name="s")
_sc_params = pltpu.CompilerParams(use_tc_tiling_on_sc=False)


def _deg_body(src_hbm, dst_hbm, ones_hbm, z1_hbm, out_hbm,
              src_v, dst_v, ones_v, dego_sh, degi_sh, sem0, sem1):
    c = lax.axis_index("c")
    s = lax.axis_index("s")
    wid = c * NS + s
    pltpu.sync_copy(src_hbm.at[wid], src_v)
    pltpu.sync_copy(dst_hbm.at[wid], dst_v)
    pltpu.sync_copy(ones_hbm, ones_v)
    sl = pl.ds(s * RPT, RPT)
    pltpu.sync_copy(z1_hbm.at[sl], dego_sh.at[sl])
    pltpu.sync_copy(z1_hbm.at[sl], degi_sh.at[sl])
    plsc.subcore_barrier()

    @pl.loop(0, CH // DGRP)
    def _(j):
        pltpu.async_copy(ones_v, dego_sh.at[src_v.at[j]], sem0, add=True)
        pltpu.async_copy(ones_v, degi_sh.at[dst_v.at[j]], sem1, add=True)

    @pl.loop(0, CH // DGRP)
    def _(j):
        pltpu.make_async_copy(ones_v, dego_sh.at[src_v.at[0]], sem0).wait()
        pltpu.make_async_copy(ones_v, degi_sh.at[dst_v.at[0]], sem1).wait()

    plsc.subcore_barrier()
    pltpu.sync_copy(dego_sh.at[sl], out_hbm.at[c, 0, sl])
    pltpu.sync_copy(degi_sh.at[sl], out_hbm.at[c, 1, sl])


_deg_call = pl.kernel(
    _deg_body,
    out_type=jax.ShapeDtypeStruct((NC, 2, N_PAD), jnp.float32),
    mesh=_mesh,
    scratch_types=[
        pltpu.VMEM((CH // DGRP, DGRP * CHUNK), jnp.int32),
        pltpu.VMEM((CH // DGRP, DGRP * CHUNK), jnp.int32),
        pltpu.VMEM((DGRP * CHUNK,), jnp.float32),
        pltpu.VMEM_SHARED((N_PAD,), jnp.float32),
        pltpu.VMEM_SHARED((N_PAD,), jnp.float32),
        pltpu.SemaphoreType.DMA,
        pltpu.SemaphoreType.DMA,
    ],
    compiler_params=_sc_params,
)


def _layer_body(h_hbm, src_hbm, dst_hbm, out_hbm,
                src_v, dst_v, rows0_v, rows1_v, h_sh, agg_sh, sem0, sem1):
    c = lax.axis_index("c")
    s = lax.axis_index("s")
    wid = c * NS + s
    sl = pl.ds(s * RPT, RPT)

    pltpu.async_copy(h_hbm.at[sl], h_sh.at[sl], sem0)
    pltpu.async_copy(src_hbm.at[wid], src_v, sem1)
    pltpu.async_copy(dst_hbm.at[wid], dst_v, sem1)

    zv = jnp.zeros((32,), jnp.bfloat16)

    @pl.loop(0, RPT)
    def _(r):
        for cb in range(D_H // 32):
            rows0_v[r, pl.ds(cb * 32, 32)] = zv

    pltpu.sync_copy(rows0_v.at[pl.ds(0, RPT)], agg_sh.at[sl])
    pltpu.make_async_copy(src_hbm.at[wid], src_v, sem1).wait()
    pltpu.make_async_copy(dst_hbm.at[wid], dst_v, sem1).wait()
    pltpu.make_async_copy(h_hbm.at[sl], h_sh.at[sl], sem0).wait()
    plsc.subcore_barrier()

    pltpu.async_copy(h_sh.at[src_v.at[0]], rows0_v, sem0)
    pltpu.async_copy(h_sh.at[src_v.at[1]], rows1_v, sem1)

    @pl.loop(0, CH // GRP // 2 - 1)
    def _(k):
        j = 2 * k
        pltpu.make_async_copy(h_sh.at[src_v.at[0]], rows0_v, sem0).wait()
        pltpu.sync_copy(rows0_v, agg_sh.at[dst_v.at[j]], add=True)
        pltpu.async_copy(h_sh.at[src_v.at[j + 2]], rows0_v, sem0)
        pltpu.make_async_copy(h_sh.at[src_v.at[1]], rows1_v, sem1).wait()
        pltpu.sync_copy(rows1_v, agg_sh.at[dst_v.at[j + 1]], add=True)
        pltpu.async_copy(h_sh.at[src_v.at[j + 3]], rows1_v, sem1)

    pltpu.make_async_copy(h_sh.at[src_v.at[0]], rows0_v, sem0).wait()
    pltpu.sync_copy(rows0_v, agg_sh.at[dst_v.at[CH // GRP - 2]], add=True)
    pltpu.make_async_copy(h_sh.at[src_v.at[1]], rows1_v, sem1).wait()
    pltpu.sync_copy(rows1_v, agg_sh.at[dst_v.at[CH // GRP - 1]], add=True)

    plsc.subcore_barrier()
    pltpu.sync_copy(agg_sh.at[sl], out_hbm.at[c, sl])


_layer_call = pl.kernel(
    _layer_body,
    out_type=jax.ShapeDtypeStruct((NC, N_PAD, D_H), jnp.bfloat16),
    mesh=_mesh,
    scratch_types=[
        pltpu.VMEM((CH // GRP, GRP * CHUNK), jnp.int32),
        pltpu.VMEM((CH // GRP, GRP * CHUNK), jnp.int32),
        pltpu.VMEM((GRP * CHUNK, D_H), jnp.bfloat16),
        pltpu.VMEM((GRP * CHUNK, D_H), jnp.bfloat16),
        pltpu.VMEM_SHARED((N_PAD, D_H), jnp.bfloat16),
        pltpu.VMEM_SHARED((N_PAD, D_H), jnp.bfloat16),
        pltpu.SemaphoreType.DMA,
        pltpu.SemaphoreType.DMA,
    ],
    compiler_params=_sc_params,
)


def _mm_body(f_ref, w_ref, xw_ref):
    xw_ref[...] = jnp.dot(f_ref[...], w_ref[...],
                          preferred_element_type=jnp.float32,
                          precision=lax.Precision.HIGHEST)


_mm_call = pl.pallas_call(
    _mm_body,
    out_shape=jax.ShapeDtypeStruct((N_PAD, D_H), jnp.float32),
)


def _prep_body(degp_ref, xw_ref, h_ref, ns_ref, nd_ref):
    dego = degp_ref[0, 0:1, :] + degp_ref[1, 0:1, :]
    degi = degp_ref[0, 1:2, :] + degp_ref[1, 1:2, :]
    ns_row = jnp.where(dego > 0.0, lax.rsqrt(jnp.maximum(dego, 1.0)), 0.0)
    nd_row = jnp.where(degi > 0.0, lax.rsqrt(jnp.maximum(degi, 1.0)), 0.0)
    ns_col = jnp.transpose(ns_row, (1, 0))
    nd_col = jnp.transpose(nd_row, (1, 0))
    ns_ref[...] = ns_col
    nd_ref[...] = nd_col
    h_ref[...] = (xw_ref[...] * ns_col).astype(jnp.bfloat16)


_prep_call = pl.pallas_call(
    _prep_body,
    out_shape=(
        jax.ShapeDtypeStruct((N_PAD, D_H), jnp.bfloat16),
        jax.ShapeDtypeStruct((N_PAD, 1), jnp.float32),
        jax.ShapeDtypeStruct((N_PAD, 1), jnp.float32),
    ),
)


def _combine_body(aggp_ref, nd_ref, ns_ref, b_ref, w_ref, h_ref):
    p = aggp_ref[0].astype(jnp.float32) + aggp_ref[1].astype(jnp.float32)
    x = jnp.maximum(p * nd_ref[...] + b_ref[...], 0.0)
    h_ref[...] = (jnp.dot(x, w_ref[...],
                          preferred_element_type=jnp.float32,
                          precision=lax.Precision.HIGHEST)
                  * ns_ref[...]).astype(jnp.bfloat16)


_combine_call = pl.pallas_call(
    _combine_body,
    out_shape=jax.ShapeDtypeStruct((N_PAD, D_H), jnp.bfloat16),
)


def _final_body(aggp_ref, nd_ref, b_ref, out_ref):
    p = aggp_ref[0].astype(jnp.float32) + aggp_ref[1].astype(jnp.float32)
    x = jnp.maximum(p * nd_ref[...] + b_ref[...], 0.0)
    out_ref[...] = jnp.sum(x[:N, :], axis=0, keepdims=True) * (1.0 / N)


_final_call = pl.pallas_call(
    _final_body,
    out_shape=jax.ShapeDtypeStruct((1, D_H), jnp.float32),
)


def kernel(features, edge_index, W1, b1, W2, b2, W3, b3):
    src = edge_index[0]
    dst = edge_index[1]
    padv = jnp.full((E_PAD - E,), N, dtype=jnp.int32)
    srcp = jnp.concatenate([src, padv]).reshape(NW, CH // GRP, GRP * CHUNK)
    dstp = jnp.concatenate([dst, padv]).reshape(NW, CH // GRP, GRP * CHUNK)
    featp = jnp.pad(features, ((0, N_PAD - N), (0, 0)))
    ones = jnp.ones((DGRP * CHUNK,), jnp.float32)
    z1 = jnp.zeros((N_PAD,), jnp.float32)
    srcd = srcp.reshape(NW, CH // DGRP, DGRP * CHUNK)
    dstd = dstp.reshape(NW, CH // DGRP, DGRP * CHUNK)

    degp = _deg_call(srcd, dstd, ones, z1)
    xw1 = _mm_call(featp, W1)
    h, ns_col, nd_col = _prep_call(degp, xw1)

    agg1 = _layer_call(h, srcp, dstp)
    h2 = _combine_call(agg1, nd_col, ns_col, b1.reshape(1, D_H), W2)
    agg2 = _layer_call(h2, srcp, dstp)
    h3 = _combine_call(agg2, nd_col, ns_col, b2.reshape(1, D_H), W3)
    agg3 = _layer_call(h3, srcp, dstp)
    out = _final_call(agg3, nd_col, b3.reshape(1, D_H))
    return out.reshape(D_H)


# speedup vs baseline: 3.4291x; 1.0119x over previous
import jax
import jax.numpy as jnp
from jax import lax
from jax.experimental import pallas as pl
from jax.experimental.pallas import tpu as pltpu
from jax.experimental.pallas import tpu_sc as plsc

N = 10000
E = 320000
D_IN = 128
D_H = 64

NC = 2
NS = 16
NW = NC * NS
CHUNK = 128
CH = 80
GRP = 8
DGRP = 16
E_PAD = NW * CH * CHUNK
N_PAD = 10240
RPT = N_PAD // NS

_mesh = plsc.VectorSubcoreMesh(core_axis_name="c", subcore_axis_name="s")
_sc_params = pltpu.CompilerParams(use_tc_tiling_on_sc=False)


def _deg_body(src_hbm, dst_hbm, ones_hbm, out_hbm,
              src_v, dst_v, ones_v, zb_v, dego_sh, degi_sh, sem0, sem1):
    c = lax.axis_index("c")
    s = lax.axis_index("s")
    wid = c * NS + s
    sl = pl.ds(s * RPT, RPT)
    pltpu.async_copy(src_hbm.at[wid], src_v, sem1)
    pltpu.async_copy(dst_hbm.at[wid], dst_v, sem1)
    pltpu.async_copy(ones_hbm, ones_v, sem0)

    zv = jnp.zeros((16,), jnp.float32)

    @pl.loop(0, RPT // 16)
    def _(r):
        zb_v[pl.ds(r * 16, 16)] = zv

    pltpu.sync_copy(zb_v, dego_sh.at[sl])
    pltpu.sync_copy(zb_v, degi_sh.at[sl])
    pltpu.make_async_copy(src_hbm.at[wid], src_v, sem1).wait()
    pltpu.make_async_copy(dst_hbm.at[wid], dst_v, sem1).wait()
    pltpu.make_async_copy(ones_hbm, ones_v, sem0).wait()
    plsc.subcore_barrier()

    @pl.loop(0, CH // DGRP)
    def _(j):
        pltpu.async_copy(ones_v, dego_sh.at[src_v.at[j]], sem0, add=True)
        pltpu.async_copy(ones_v, degi_sh.at[dst_v.at[j]], sem1, add=True)

    @pl.loop(0, CH // DGRP)
    def _(j):
        pltpu.make_async_copy(ones_v, dego_sh.at[src_v.at[0]], sem0).wait()
        pltpu.make_async_copy(ones_v, degi_sh.at[dst_v.at[0]], sem1).wait()

    plsc.subcore_barrier()
    pltpu.sync_copy(dego_sh.at[sl], out_hbm.at[c, 0, sl])
    pltpu.sync_copy(degi_sh.at[sl], out_hbm.at[c, 1, sl])


_deg_call = pl.kernel(
    _deg_body,
    out_type=jax.ShapeDtypeStruct((NC, 2, N_PAD), jnp.float32),
    mesh=_mesh,
    scratch_types=[
        pltpu.VMEM((CH // DGRP, DGRP * CHUNK), jnp.int32),
        pltpu.VMEM((CH // DGRP, DGRP * CHUNK), jnp.int32),
        pltpu.VMEM((DGRP * CHUNK,), jnp.float32),
        pltpu.VMEM((RPT,), jnp.float32),
        pltpu.VMEM_SHARED((N_PAD,), jnp.float32),
        pltpu.VMEM_SHARED((N_PAD,), jnp.float32),
        pltpu.SemaphoreType.DMA,
        pltpu.SemaphoreType.DMA,
    ],
    compiler_params=_sc_params,
)


def _layer_body(h_hbm, src_hbm, dst_hbm, out_hbm,
                src_v, dst_v, rows0_v, rows1_v, h_sh, agg_sh, sem0, sem1):
    c = lax.axis_index("c")
    s = lax.axis_index("s")
    wid = c * NS + s
    sl = pl.ds(s * RPT, RPT)

    pltpu.async_copy(h_hbm.at[sl], h_sh.at[sl], sem0)
    pltpu.async_copy(src_hbm.at[wid], src_v, sem1)
    pltpu.async_copy(dst_hbm.at[wid], dst_v, sem1)

    zv = jnp.zeros((32,), jnp.bfloat16)

    @pl.loop(0, RPT)
    def _(r):
        for cb in range(D_H // 32):
            rows0_v[r, pl.ds(cb * 32, 32)] = zv

    pltpu.sync_copy(rows0_v.at[pl.ds(0, RPT)], agg_sh.at[sl])
    pltpu.make_async_copy(src_hbm.at[wid], src_v, sem1).wait()
    pltpu.make_async_copy(dst_hbm.at[wid], dst_v, sem1).wait()
    pltpu.make_async_copy(h_hbm.at[sl], h_sh.at[sl], sem0).wait()
    plsc.subcore_barrier()

    pltpu.async_copy(h_sh.at[src_v.at[0]], rows0_v, sem0)
    pltpu.async_copy(h_sh.at[src_v.at[1]], rows1_v, sem1)

    @pl.loop(0, CH // GRP // 2 - 1)
    def _(k):
        j = 2 * k
        pltpu.make_async_copy(h_sh.at[src_v.at[0]], rows0_v, sem0).wait()
        pltpu.sync_copy(rows0_v, agg_sh.at[dst_v.at[j]], add=True)
        pltpu.async_copy(h_sh.at[src_v.at[j + 2]], rows0_v, sem0)
        pltpu.make_async_copy(h_sh.at[src_v.at[1]], rows1_v, sem1).wait()
        pltpu.sync_copy(rows1_v, agg_sh.at[dst_v.at[j + 1]], add=True)
        pltpu.async_copy(h_sh.at[src_v.at[j + 3]], rows1_v, sem1)

    pltpu.make_async_copy(h_sh.at[src_v.at[0]], rows0_v, sem0).wait()
    pltpu.sync_copy(rows0_v, agg_sh.at[dst_v.at[CH // GRP - 2]], add=True)
    pltpu.make_async_copy(h_sh.at[src_v.at[1]], rows1_v, sem1).wait()
    pltpu.sync_copy(rows1_v, agg_sh.at[dst_v.at[CH // GRP - 1]], add=True)

    plsc.subcore_barrier()
    pltpu.sync_copy(agg_sh.at[sl], out_hbm.at[c, sl])


_layer_call = pl.kernel(
    _layer_body,
    out_type=jax.ShapeDtypeStruct((NC, N_PAD, D_H), jnp.bfloat16),
    mesh=_mesh,
    scratch_types=[
        pltpu.VMEM((CH // GRP, GRP * CHUNK), jnp.int32),
        pltpu.VMEM((CH // GRP, GRP * CHUNK), jnp.int32),
        pltpu.VMEM((GRP * CHUNK, D_H), jnp.bfloat16),
        pltpu.VMEM((GRP * CHUNK, D_H), jnp.bfloat16),
        pltpu.VMEM_SHARED((N_PAD, D_H), jnp.bfloat16),
        pltpu.VMEM_SHARED((N_PAD, D_H), jnp.bfloat16),
        pltpu.SemaphoreType.DMA,
        pltpu.SemaphoreType.DMA,
    ],
    compiler_params=_sc_params,
)


def _mm_body(f_ref, w_ref, xw_ref):
    xw_ref[...] = jnp.dot(f_ref[...], w_ref[...],
                          preferred_element_type=jnp.float32,
                          precision=lax.Precision.HIGHEST)


_mm_call = pl.pallas_call(
    _mm_body,
    out_shape=jax.ShapeDtypeStruct((N_PAD, D_H), jnp.float32),
)


def _prep_body(degp_ref, xw_ref, h_ref, ns_ref, nd_ref):
    dego = degp_ref[0, 0:1, :] + degp_ref[1, 0:1, :]
    degi = degp_ref[0, 1:2, :] + degp_ref[1, 1:2, :]
    ns_row = jnp.where(dego > 0.0, lax.rsqrt(jnp.maximum(dego, 1.0)), 0.0)
    nd_row = jnp.where(degi > 0.0, lax.rsqrt(jnp.maximum(degi, 1.0)), 0.0)
    ns_col = jnp.transpose(ns_row, (1, 0))
    nd_col = jnp.transpose(nd_row, (1, 0))
    ns_ref[...] = ns_col
    nd_ref[...] = nd_col
    h_ref[...] = (xw_ref[...] * ns_col).astype(jnp.bfloat16)


_prep_call = pl.pallas_call(
    _prep_body,
    out_shape=(
        jax.ShapeDtypeStruct((N_PAD, D_H), jnp.bfloat16),
        jax.ShapeDtypeStruct((N_PAD, 1), jnp.float32),
        jax.ShapeDtypeStruct((N_PAD, 1), jnp.float32),
    ),
)


def _combine_body(aggp_ref, nd_ref, ns_ref, b_ref, w_ref, h_ref):
    p = aggp_ref[0].astype(jnp.float32) + aggp_ref[1].astype(jnp.float32)
    x = jnp.maximum(p * nd_ref[...] + b_ref[...], 0.0)
    h_ref[...] = (jnp.dot(x, w_ref[...],
                          preferred_element_type=jnp.float32,
                          precision=lax.Precision.HIGHEST)
                  * ns_ref[...]).astype(jnp.bfloat16)


_combine_call = pl.pallas_call(
    _combine_body,
    out_shape=jax.ShapeDtypeStruct((N_PAD, D_H), jnp.bfloat16),
)


def _final_body(aggp_ref, nd_ref, b_ref, out_ref):
    p = aggp_ref[0].astype(jnp.float32) + aggp_ref[1].astype(jnp.float32)
    x = jnp.maximum(p * nd_ref[...] + b_ref[...], 0.0)
    out_ref[...] = jnp.sum(x[:N, :], axis=0, keepdims=True) * (1.0 / N)


_final_call = pl.pallas_call(
    _final_body,
    out_shape=jax.ShapeDtypeStruct((1, D_H), jnp.float32),
)


def kernel(features, edge_index, W1, b1, W2, b2, W3, b3):
    src = edge_index[0]
    dst = edge_index[1]
    padv = jnp.full((E_PAD - E,), N, dtype=jnp.int32)
    srcp = jnp.concatenate([src, padv]).reshape(NW, CH // GRP, GRP * CHUNK)
    dstp = jnp.concatenate([dst, padv]).reshape(NW, CH // GRP, GRP * CHUNK)
    featp = jnp.pad(features, ((0, N_PAD - N), (0, 0)))
    ones = jnp.ones((DGRP * CHUNK,), jnp.float32)
    srcd = srcp.reshape(NW, CH // DGRP, DGRP * CHUNK)
    dstd = dstp.reshape(NW, CH // DGRP, DGRP * CHUNK)

    degp = _deg_call(srcd, dstd, ones)
    xw1 = _mm_call(featp, W1)
    h, ns_col, nd_col = _prep_call(degp, xw1)

    agg1 = _layer_call(h, srcp, dstp)
    h2 = _combine_call(agg1, nd_col, ns_col, b1.reshape(1, D_H), W2)
    agg2 = _layer_call(h2, srcp, dstp)
    h3 = _combine_call(agg2, nd_col, ns_col, b2.reshape(1, D_H), W3)
    agg3 = _layer_call(h3, srcp, dstp)
    out = _final_call(agg3, nd_col, b3.reshape(1, D_H))
    return out.reshape(D_H)
